# baseline (speedup 1.0000x reference)
import functools
import numpy as np
import jax
import jax.numpy as jnp
from jax import lax
from jax.experimental import pallas as pl
from jax.experimental.pallas import tpu as pltpu
from jax.experimental.pallas import tpu_sc as plsc

N_BINS = 18
B = 32
T = 16384
TC_CHUNK = 2048
N_CHUNKS = T // TC_CHUNK
ACC = (N_BINS + 1) * B

_INV_DELTA = np.float32((N_BINS) / (2.0 * np.pi))
_PI = np.float32(np.pi)


def _sc_body(pha_hbm, ampT_hbm, out_hbm, pha_v, amp_v, offs_v, acc_v):
    j = lax.axis_index("s") * 2 + lax.axis_index("c")

    def zero(k, _):
        acc_v[pl.ds(k * 16, 16)] = jnp.zeros((16,), jnp.float32)
        return 0

    lax.fori_loop(0, ACC // 16, zero, 0, unroll=True)

    def chunk_body(c, _):
        pltpu.sync_copy(pha_hbm.at[pl.ds(j * T + c * TC_CHUNK, TC_CHUNK)], pha_v)
        pltpu.sync_copy(ampT_hbm.at[pl.ds(c * TC_CHUNK * B, TC_CHUNK * B)], amp_v)

        def binify(k, _):
            v = pha_v[pl.ds(k * 16, 16)]
            f = (v + _PI) * _INV_DELTA
            idx = f.astype(jnp.int32)
            idx = jnp.minimum(idx, N_BINS)
            idx = jnp.where(f < 0.0, N_BINS, idx)
            offs_v[pl.ds(k * 16, 16)] = idx * B
            return 0

        lax.fori_loop(0, TC_CHUNK // 16, binify, 0, unroll=4)

        def accum(g, _):
            off_vec = offs_v[pl.ds(g * 16, 16)]
            base = g * (16 * B)
            for u in range(16):
                off = off_vec[u]
                a0 = amp_v[pl.ds(base + u * B, 16)]
                a1 = amp_v[pl.ds(base + u * B + 16, 16)]
                plsc.addupdate(acc_v.at[pl.ds(off, 16)], a0)
                plsc.addupdate(acc_v.at[pl.ds(off + 16, 16)], a1)
            return 0

        lax.fori_loop(0, TC_CHUNK // 16, accum, 0)
        return 0

    lax.fori_loop(0, N_CHUNKS, chunk_body, 0)
    pltpu.sync_copy(acc_v, out_hbm.at[pl.ds(j * ACC, ACC)])


@functools.partial(
    pl.kernel,
    out_type=jax.ShapeDtypeStruct((B * ACC,), jnp.float32),
    mesh=plsc.VectorSubcoreMesh(
        core_axis_name="c", subcore_axis_name="s", num_cores=2, num_subcores=16
    ),
    scratch_types=[
        pltpu.VMEM((TC_CHUNK,), jnp.float32),
        pltpu.VMEM((TC_CHUNK * B,), jnp.float32),
        pltpu.VMEM((TC_CHUNK,), jnp.int32),
        pltpu.VMEM((ACC,), jnp.float32),
    ],
)
def _sc_binsum(pha_hbm, ampT_hbm, out_hbm, pha_v, amp_v, offs_v, acc_v):
    _sc_body(pha_hbm, ampT_hbm, out_hbm, pha_v, amp_v, offs_v, acc_v)


def _entropy_body(sums_ref, out_ref):
    s = sums_ref[...]
    s18 = s[:, :N_BINS, :]
    tot = jnp.sum(s18, axis=1, keepdims=True)
    p = s18 / tot
    inv_log_n = np.float32(1.0 / np.log(float(N_BINS)))
    mi = 1.0 + inv_log_n * jnp.sum(p * jnp.log(p), axis=1)
    out_ref[...] = mi


@jax.jit
def kernel(pha, amp):
    pha_flat = pha.reshape(-1)
    ampT_flat = amp.T.reshape(-1)
    sums = _sc_binsum(pha_flat, ampT_flat)
    mit = pl.pallas_call(
        _entropy_body,
        out_shape=jax.ShapeDtypeStruct((B, B), jnp.float32),
    )(sums.reshape(B, N_BINS + 1, B))
    return mit.T
